# baseline (speedup 1.0000x reference)
import jax
import jax.numpy as jnp
from jax.experimental import pallas as pl

_BM = 400
_BA = 2000


def _xw1_body(x_ref, w1_ref, out_ref):
    out_ref[...] = jnp.dot(x_ref[...], w1_ref[...],
                           preferred_element_type=jnp.float32)


def _pass1_body(adj_ref, s1_ref, w2_ref, out_ref):
    acc = jnp.dot(adj_ref[...], s1_ref[...],
                  preferred_element_type=jnp.float32)
    h = jnp.maximum(acc, 0.0)
    out_ref[...] = jnp.dot(h, w2_ref[...],
                           preferred_element_type=jnp.float32)


def _pass2_body(adj_ref, s2_ref, out_ref):
    acc = jnp.dot(adj_ref[...], s2_ref[...],
                  preferred_element_type=jnp.float32)
    out_ref[...] = jnp.maximum(acc, 0.0)


def _abar_body(zi_ref, zj_ref, out_ref):
    out_ref[...] = jax.lax.dot_general(
        zi_ref[...], zj_ref[...],
        (((1,), (1,)), ((), ())),
        preferred_element_type=jnp.float32)


def kernel(x, adj, W1, W2):
    n, d_in = x.shape
    d_h1 = W1.shape[1]
    d_z = W2.shape[1]

    s1 = pl.pallas_call(
        _xw1_body,
        grid=(n // _BA,),
        in_specs=[
            pl.BlockSpec((_BA, d_in), lambda i: (i, 0)),
            pl.BlockSpec((d_in, d_h1), lambda i: (0, 0)),
        ],
        out_specs=pl.BlockSpec((_BA, d_h1), lambda i: (i, 0)),
        out_shape=jax.ShapeDtypeStruct((n, d_h1), jnp.float32),
    )(x, W1)

    s2 = pl.pallas_call(
        _pass1_body,
        grid=(n // _BM,),
        in_specs=[
            pl.BlockSpec((_BM, n), lambda i: (i, 0)),
            pl.BlockSpec((n, d_h1), lambda i: (0, 0)),
            pl.BlockSpec((d_h1, d_z), lambda i: (0, 0)),
        ],
        out_specs=pl.BlockSpec((_BM, d_z), lambda i: (i, 0)),
        out_shape=jax.ShapeDtypeStruct((n, d_z), jnp.float32),
    )(adj, s1, W2)

    z = pl.pallas_call(
        _pass2_body,
        grid=(n // _BM,),
        in_specs=[
            pl.BlockSpec((_BM, n), lambda i: (i, 0)),
            pl.BlockSpec((n, d_z), lambda i: (0, 0)),
        ],
        out_specs=pl.BlockSpec((_BM, d_z), lambda i: (i, 0)),
        out_shape=jax.ShapeDtypeStruct((n, d_z), jnp.float32),
    )(adj, s2)

    a_bar = pl.pallas_call(
        _abar_body,
        grid=(n // _BM,),
        in_specs=[
            pl.BlockSpec((_BM, d_z), lambda i: (i, 0)),
            pl.BlockSpec((n, d_z), lambda i: (0, 0)),
        ],
        out_specs=pl.BlockSpec((_BM, n), lambda i: (i, 0)),
        out_shape=jax.ShapeDtypeStruct((n, n), jnp.float32),
    )(z, z)

    return (a_bar, z)
